# Initial kernel scaffold; baseline (speedup 1.0000x reference)
import functools

import jax
import jax.numpy as jnp
from jax import lax
from jax.experimental import pallas as pl
from jax.experimental.pallas import tpu as pltpu
from jax.experimental.pallas import tpu_sc as plsc

N_CORES = 2
N_SUB = 16
N_WORKERS = N_CORES * N_SUB
LANES = 16
EDGE_BLK = 80


@functools.lru_cache(maxsize=None)
def _build_sc_aggregate(n, d, e):
    per_worker = e // N_WORKERS
    assert per_worker * N_WORKERS == e
    n_win = per_worker // EDGE_BLK
    assert n_win * EDGE_BLK == per_worker
    rows_per_tile = n // N_SUB
    assert rows_per_tile * N_SUB == n
    zchunk = 125
    n_zc = rows_per_tile // zchunk
    assert n_zc * zchunk == rows_per_tile

    mesh = plsc.VectorSubcoreMesh(core_axis_name="c", subcore_axis_name="s")

    @functools.partial(
        pl.kernel,
        out_type=jax.ShapeDtypeStruct((N_CORES, n, d), jnp.float32),
        mesh=mesh,
        scratch_types=[
            pltpu.VMEM((EDGE_BLK,), jnp.int32),
            pltpu.VMEM((EDGE_BLK,), jnp.int32),
            pltpu.VMEM((EDGE_BLK,), jnp.float32),
            pltpu.VMEM((EDGE_BLK, d), jnp.float32),
            pltpu.VMEM((125, d), jnp.float32),
            pltpu.VMEM_SHARED((n, d), jnp.float32),
            pltpu.SemaphoreType.DMA,
        ],
    )
    def sc_aggr(h_hbm, src_hbm, dst_hbm, w_hbm, out_hbm,
                src_v, dst_v, w_v, rows_v, zbuf, acc_sp, sem):
        c = lax.axis_index("c")
        s = lax.axis_index("s")
        wid = c * N_SUB + s
        zchunk_ = 125

        def z_body(i, _):
            for j in range(d // LANES):
                zbuf[i, pl.ds(j * LANES, LANES)] = jnp.zeros((LANES,), jnp.float32)
            return 0
        lax.fori_loop(0, zchunk_, z_body, 0)
        row0 = s * rows_per_tile
        for k in range(n_zc):
            pltpu.sync_copy(zbuf, acc_sp.at[pl.ds(row0 + k * zchunk_, zchunk_)])
        plsc.subcore_barrier()

        ebase = wid * per_worker

        def win_body(win, _):
            base = ebase + win * EDGE_BLK
            pltpu.sync_copy(src_hbm.at[pl.ds(base, EDGE_BLK)], src_v)
            pltpu.sync_copy(dst_hbm.at[pl.ds(base, EDGE_BLK)], dst_v)
            pltpu.sync_copy(w_hbm.at[pl.ds(base, EDGE_BLK)], w_v)
            pltpu.async_copy(h_hbm.at[src_v], rows_v, sem).wait()

            def g_body(g, _):
                gb = g * LANES
                w16 = w_v[pl.ds(gb, LANES)]
                for b in range(LANES):
                    wb = jnp.take(w16, jnp.full((LANES,), b, jnp.int32), axis=0,
                                  mode="promise_in_bounds")
                    for j in range(d // LANES):
                        sl = pl.ds(j * LANES, LANES)
                        rows_v[gb + b, sl] = rows_v[gb + b, sl] * wb
                return 0
            lax.fori_loop(0, EDGE_BLK // LANES, g_body, 0)

            pltpu.sync_copy(rows_v, acc_sp.at[dst_v], add=True)
            return 0
        lax.fori_loop(0, n_win, win_body, 0)

        plsc.subcore_barrier()
        for k in range(n_zc):
            r = row0 + k * zchunk_
            pltpu.sync_copy(acc_sp.at[pl.ds(r, zchunk_)], zbuf)
            pltpu.sync_copy(zbuf, out_hbm.at[c, pl.ds(r, zchunk_)])

    return sc_aggr


@functools.lru_cache(maxsize=None)
def _build_tc_update(n, d):
    blk = 1000
    assert n % blk == 0

    def tc_body(a0_ref, a1_ref, h_ref, w1t_ref, w2t_ref, b_ref, out_ref):
        aggr = a0_ref[...] + a1_ref[...]
        z = jnp.dot(aggr, w1t_ref[...], preferred_element_type=jnp.float32)
        z = z + jnp.dot(h_ref[...], w2t_ref[...],
                        preferred_element_type=jnp.float32)
        out_ref[...] = jnp.tanh(z + b_ref[...])

    return pl.pallas_call(
        tc_body,
        grid=(n // blk,),
        in_specs=[
            pl.BlockSpec((blk, d), lambda i: (i, 0)),
            pl.BlockSpec((blk, d), lambda i: (i, 0)),
            pl.BlockSpec((blk, d), lambda i: (i, 0)),
            pl.BlockSpec((d, d), lambda i: (0, 0)),
            pl.BlockSpec((d, d), lambda i: (0, 0)),
            pl.BlockSpec((1, d), lambda i: (0, 0)),
        ],
        out_specs=pl.BlockSpec((blk, d), lambda i: (i, 0)),
        out_shape=jax.ShapeDtypeStruct((n, d), jnp.float32),
    )


def kernel(x, edge_index, edge_weights, Ws, bs):
    n, d = x.shape
    e = edge_weights.shape[0]
    num_layers = Ws.shape[0]

    src = edge_index[0]
    dst = edge_index[1]
    w1t = jnp.transpose(Ws[:, :, :d], (0, 2, 1))
    w2t = jnp.transpose(Ws[:, :, d:], (0, 2, 1))
    b2 = bs.reshape(num_layers, 1, d)

    sc_aggr = _build_sc_aggregate(n, d, e)
    tc_update = _build_tc_update(n, d)

    h = x
    for l in range(num_layers):
        parts = sc_aggr(h, src, dst, edge_weights)
        h = tc_update(parts[0], parts[1], h, w1t[l], w2t[l], b2[l])
    return h


# speedup vs baseline: 3.6301x; 3.6301x over previous
import functools

import jax
import jax.numpy as jnp
from jax import lax
from jax.experimental import pallas as pl
from jax.experimental.pallas import tpu as pltpu
from jax.experimental.pallas import tpu_sc as plsc

N_CORES = 2
N_SUB = 16
N_WORKERS = N_CORES * N_SUB
LANES = 16
EDGE_BLK = 80


@functools.lru_cache(maxsize=None)
def _build_sc_aggregate(n, d, e):
    per_worker = e // N_WORKERS
    assert per_worker * N_WORKERS == e
    n_win = per_worker // EDGE_BLK
    assert n_win * EDGE_BLK == per_worker
    rows_per_tile = n // N_SUB
    assert rows_per_tile * N_SUB == n
    zchunk = 128
    n_zc = rows_per_tile // zchunk
    assert n_zc * zchunk == rows_per_tile

    mesh = plsc.VectorSubcoreMesh(core_axis_name="c", subcore_axis_name="s")

    @functools.partial(
        pl.kernel,
        out_type=jax.ShapeDtypeStruct((N_CORES, n, d), jnp.float32),
        mesh=mesh,
        scratch_types=[
            pltpu.VMEM((EDGE_BLK,), jnp.int32),
            pltpu.VMEM((EDGE_BLK,), jnp.int32),
            pltpu.VMEM((EDGE_BLK,), jnp.float32),
            pltpu.VMEM((EDGE_BLK, d), jnp.float32),
            pltpu.VMEM((128, d), jnp.float32),
            pltpu.VMEM_SHARED((n, d), jnp.float32),
            pltpu.SemaphoreType.DMA,
        ],
    )
    def sc_aggr(h_hbm, src_hbm, dst_hbm, w_hbm, out_hbm,
                src_v, dst_v, w_v, rows_v, zbuf, acc_sp, sem):
        c = lax.axis_index("c")
        s = lax.axis_index("s")
        wid = c * N_SUB + s
        zchunk_ = 128

        def z_body(i, _):
            for j in range(d // LANES):
                zbuf[i, pl.ds(j * LANES, LANES)] = jnp.zeros((LANES,), jnp.float32)
            return 0
        lax.fori_loop(0, zchunk_, z_body, 0)
        row0 = s * rows_per_tile
        for k in range(n_zc):
            pltpu.sync_copy(zbuf, acc_sp.at[pl.ds(row0 + k * zchunk_, zchunk_)])
        plsc.subcore_barrier()

        ebase = wid * per_worker

        def win_body(win, _):
            base = ebase + win * EDGE_BLK
            pltpu.sync_copy(src_hbm.at[pl.ds(base, EDGE_BLK)], src_v)
            pltpu.sync_copy(dst_hbm.at[pl.ds(base, EDGE_BLK)], dst_v)
            pltpu.sync_copy(w_hbm.at[pl.ds(base, EDGE_BLK)], w_v)
            pltpu.async_copy(h_hbm.at[src_v], rows_v, sem).wait()

            def g_body(g, _):
                gb = g * LANES
                w16 = w_v[pl.ds(gb, LANES)]
                for b in range(LANES):
                    wb = lax.gather(
                        w16, jnp.full((LANES, 1), b, jnp.int32),
                        lax.GatherDimensionNumbers(
                            offset_dims=(), collapsed_slice_dims=(0,),
                            start_index_map=(0,)),
                        slice_sizes=(1,),
                        mode=lax.GatherScatterMode.PROMISE_IN_BOUNDS)
                    for j in range(d // LANES):
                        sl = pl.ds(j * LANES, LANES)
                        rows_v[gb + b, sl] = rows_v[gb + b, sl] * wb
                return 0
            lax.fori_loop(0, EDGE_BLK // LANES, g_body, 0)

            pltpu.sync_copy(rows_v, acc_sp.at[dst_v], add=True)
            return 0
        lax.fori_loop(0, n_win, win_body, 0)

        plsc.subcore_barrier()
        for k in range(n_zc):
            r = row0 + k * zchunk_
            pltpu.sync_copy(acc_sp.at[pl.ds(r, zchunk_)], zbuf)
            pltpu.sync_copy(zbuf, out_hbm.at[c, pl.ds(r, zchunk_)])

    return sc_aggr


@functools.lru_cache(maxsize=None)
def _build_tc_update(n, d):
    blk = 1024
    assert n % blk == 0

    def tc_body(a0_ref, a1_ref, h_ref, w1t_ref, w2t_ref, b_ref, out_ref):
        aggr = a0_ref[...] + a1_ref[...]
        z = jnp.dot(aggr, w1t_ref[...], preferred_element_type=jnp.float32)
        z = z + jnp.dot(h_ref[...], w2t_ref[...],
                        preferred_element_type=jnp.float32)
        out_ref[...] = jnp.tanh(z + b_ref[...])

    return pl.pallas_call(
        tc_body,
        grid=(n // blk,),
        in_specs=[
            pl.BlockSpec((blk, d), lambda i: (i, 0)),
            pl.BlockSpec((blk, d), lambda i: (i, 0)),
            pl.BlockSpec((blk, d), lambda i: (i, 0)),
            pl.BlockSpec((d, d), lambda i: (0, 0)),
            pl.BlockSpec((d, d), lambda i: (0, 0)),
            pl.BlockSpec((1, d), lambda i: (0, 0)),
        ],
        out_specs=pl.BlockSpec((blk, d), lambda i: (i, 0)),
        out_shape=jax.ShapeDtypeStruct((n, d), jnp.float32),
    )


def kernel(x, edge_index, edge_weights, Ws, bs):
    n, d = x.shape
    e = edge_weights.shape[0]
    num_layers = Ws.shape[0]

    n_pad = ((n + 2047) // 2048) * 2048
    src = edge_index[0]
    dst = edge_index[1]
    w1t = jnp.transpose(Ws[:, :, :d], (0, 2, 1))
    w2t = jnp.transpose(Ws[:, :, d:], (0, 2, 1))
    b2 = bs.reshape(num_layers, 1, d)

    sc_aggr = _build_sc_aggregate(n_pad, d, e)
    tc_update = _build_tc_update(n_pad, d)

    h = jnp.pad(x, ((0, n_pad - n), (0, 0)))
    for l in range(num_layers):
        parts = sc_aggr(h, src, dst, edge_weights)
        h = tc_update(parts[0], parts[1], h, w1t[l], w2t[l], b2[l])
    return h[:n]


# speedup vs baseline: 6.0719x; 1.6726x over previous
import functools

import jax
import jax.numpy as jnp
from jax import lax
from jax.experimental import pallas as pl
from jax.experimental.pallas import tpu as pltpu
from jax.experimental.pallas import tpu_sc as plsc

N_CORES = 2
N_SUB = 16
N_WORKERS = N_CORES * N_SUB
LANES = 16
SB = 128


@functools.lru_cache(maxsize=None)
def _build_sc_aggregate(n, d, e_pad):
    per_worker = e_pad // N_WORKERS
    assert per_worker * N_WORKERS == e_pad
    n_sb = per_worker // SB
    assert n_sb * SB == per_worker
    rows_per_tile = n // N_SUB
    zchunk = 128
    n_zc = rows_per_tile // zchunk
    assert n_zc * zchunk == rows_per_tile

    mesh = plsc.VectorSubcoreMesh(core_axis_name="c", subcore_axis_name="s")

    @functools.partial(
        pl.kernel,
        out_type=jax.ShapeDtypeStruct((N_CORES, n, d), jnp.float32),
        mesh=mesh,
        scratch_types=[
            pltpu.VMEM((SB,), jnp.int32),
            pltpu.VMEM((SB,), jnp.int32),
            pltpu.VMEM((SB,), jnp.int32),
            pltpu.VMEM((SB,), jnp.int32),
            pltpu.VMEM((SB,), jnp.float32),
            pltpu.VMEM((SB,), jnp.float32),
            pltpu.VMEM((SB, d), jnp.float32),
            pltpu.VMEM((SB, d), jnp.float32),
            pltpu.VMEM_SHARED((n, d), jnp.float32),
            pltpu.SemaphoreType.DMA,
            pltpu.SemaphoreType.DMA,
            pltpu.SemaphoreType.DMA,
            pltpu.SemaphoreType.DMA,
        ],
    )
    def sc_aggr(h_hbm, src_hbm, dst_hbm, w_hbm, out_hbm,
                src_a, src_b, dst_a, dst_b, w_a, w_b, rows_a, rows_b,
                acc_sp, gsem_a, gsem_b, ssem_a, ssem_b):
        zbuf = rows_a
        c = lax.axis_index("c")
        s = lax.axis_index("s")
        wid = c * N_SUB + s
        bufs = ((src_a, dst_a, w_a, rows_a, gsem_a, ssem_a),
                (src_b, dst_b, w_b, rows_b, gsem_b, ssem_b))

        def z_body(i, _):
            for j in range(d // LANES):
                zbuf[i, pl.ds(j * LANES, LANES)] = jnp.zeros((LANES,), jnp.float32)
            return 0
        lax.fori_loop(0, 128, z_body, 0)
        row0 = s * rows_per_tile
        for k in range(n_zc):
            pltpu.sync_copy(zbuf, acc_sp.at[pl.ds(row0 + k * 128, zchunk)])
        plsc.subcore_barrier()

        ebase = wid * per_worker

        def load_and_gather(i, q):
            src_v, dst_v, w_v, rows_v, gsem, _ = bufs[q]
            base = ebase + i * SB
            pltpu.sync_copy(src_hbm.at[pl.ds(base, SB)], src_v)
            pltpu.sync_copy(dst_hbm.at[pl.ds(base, SB)], dst_v)
            pltpu.sync_copy(w_hbm.at[pl.ds(base, SB)], w_v)
            pltpu.async_copy(h_hbm.at[src_v], rows_v, gsem)

        def drain_gather(q):
            src_v, _, _, rows_v, gsem, _ = bufs[q]
            pltpu.make_async_copy(h_hbm.at[src_v], rows_v, gsem).wait()

        def scale(q):
            _, _, w_v, rows_v, _, _ = bufs[q]

            def g_body(g, _):
                gb = g * LANES
                w16 = w_v[pl.ds(gb, LANES)]
                for b in range(LANES):
                    wb = lax.gather(
                        w16, jnp.full((LANES, 1), b, jnp.int32),
                        lax.GatherDimensionNumbers(
                            offset_dims=(), collapsed_slice_dims=(0,),
                            start_index_map=(0,)),
                        slice_sizes=(1,),
                        mode=lax.GatherScatterMode.PROMISE_IN_BOUNDS)
                    for j in range(d // LANES):
                        sl = pl.ds(j * LANES, LANES)
                        rows_v[gb + b, sl] = rows_v[gb + b, sl] * wb
                return 0
            lax.fori_loop(0, SB // LANES, g_body, 0)

        def start_scatter(q):
            _, dst_v, _, rows_v, _, ssem = bufs[q]
            pltpu.async_copy(rows_v, acc_sp.at[dst_v], ssem, add=True)

        def drain_scatter(q):
            _, dst_v, _, rows_v, _, ssem = bufs[q]
            pltpu.make_async_copy(rows_v, acc_sp.at[dst_v], ssem).wait()

        load_and_gather(0, 0)

        def sb_body(i, _):
            p = lax.rem(i, 2)
            for q in range(2):
                @pl.when(p == q)
                def _():
                    drain_gather(q)

                    @pl.when(i < n_sb - 1)
                    def _():
                        @pl.when(i >= 1)
                        def _():
                            drain_scatter(1 - q)
                        load_and_gather(i + 1, 1 - q)
                    scale(q)
                    start_scatter(q)
            return 0
        lax.fori_loop(0, n_sb, sb_body, 0)
        drain_scatter((n_sb - 1) % 2)
        if n_sb >= 2:
            drain_scatter(n_sb % 2)

        plsc.subcore_barrier()
        for k in range(n_zc):
            r = row0 + k * 128
            pltpu.sync_copy(acc_sp.at[pl.ds(r, zchunk)], zbuf)
            pltpu.sync_copy(zbuf, out_hbm.at[c, pl.ds(r, zchunk)])

    return sc_aggr


@functools.lru_cache(maxsize=None)
def _build_tc_update(n, d):
    blk = 1024
    assert n % blk == 0

    def tc_body(a0_ref, a1_ref, h_ref, w1t_ref, w2t_ref, b_ref, out_ref):
        aggr = a0_ref[...] + a1_ref[...]
        z = jnp.dot(aggr, w1t_ref[...], preferred_element_type=jnp.float32)
        z = z + jnp.dot(h_ref[...], w2t_ref[...],
                        preferred_element_type=jnp.float32)
        out_ref[...] = jnp.tanh(z + b_ref[...])

    return pl.pallas_call(
        tc_body,
        grid=(n // blk,),
        in_specs=[
            pl.BlockSpec((blk, d), lambda i: (i, 0)),
            pl.BlockSpec((blk, d), lambda i: (i, 0)),
            pl.BlockSpec((blk, d), lambda i: (i, 0)),
            pl.BlockSpec((d, d), lambda i: (0, 0)),
            pl.BlockSpec((d, d), lambda i: (0, 0)),
            pl.BlockSpec((1, d), lambda i: (0, 0)),
        ],
        out_specs=pl.BlockSpec((blk, d), lambda i: (i, 0)),
        out_shape=jax.ShapeDtypeStruct((n, d), jnp.float32),
    )


def kernel(x, edge_index, edge_weights, Ws, bs):
    n, d = x.shape
    e = edge_weights.shape[0]
    num_layers = Ws.shape[0]

    n_pad = ((n + 2047) // 2048) * 2048
    chunk = N_WORKERS * SB
    e_pad = ((e + chunk - 1) // chunk) * chunk

    fill = (jnp.arange(e_pad - e, dtype=jnp.int32) % n_pad)
    src = jnp.concatenate([edge_index[0], fill])
    dst = jnp.concatenate([edge_index[1], fill])
    ew = jnp.concatenate(
        [edge_weights, jnp.zeros((e_pad - e,), jnp.float32)])

    w1t = jnp.transpose(Ws[:, :, :d], (0, 2, 1))
    w2t = jnp.transpose(Ws[:, :, d:], (0, 2, 1))
    b2 = bs.reshape(num_layers, 1, d)

    sc_aggr = _build_sc_aggregate(n_pad, d, e_pad)
    tc_update = _build_tc_update(n_pad, d)

    h = jnp.pad(x, ((0, n_pad - n), (0, 0)))
    for l in range(num_layers):
        parts = sc_aggr(h, src, dst, ew)
        h = tc_update(parts[0], parts[1], h, w1t[l], w2t[l], b2[l])
    return h[:n]


# speedup vs baseline: 9.9365x; 1.6365x over previous
import functools

import jax
import jax.numpy as jnp
from jax import lax
from jax.experimental import pallas as pl
from jax.experimental.pallas import tpu as pltpu
from jax.experimental.pallas import tpu_sc as plsc

N_CORES = 2
N_SUB = 16
N_WORKERS = N_CORES * N_SUB
LANES = 16
SB = 96


@functools.lru_cache(maxsize=None)
def _build_sc_aggregate(n, d, e_pad):
    per_worker = e_pad // N_WORKERS
    assert per_worker * N_WORKERS == e_pad
    n_sb = per_worker // SB
    assert n_sb * SB == per_worker and n_sb >= 2
    rows_per_tile = n // N_SUB
    zchunk = 80
    n_zc = rows_per_tile // zchunk
    assert n_zc * zchunk == rows_per_tile and zchunk <= SB

    mesh = plsc.VectorSubcoreMesh(core_axis_name="c", subcore_axis_name="s")

    @functools.partial(
        pl.kernel,
        out_type=jax.ShapeDtypeStruct((N_CORES, n, d), jnp.float32),
        mesh=mesh,
        scratch_types=[
            pltpu.VMEM((per_worker,), jnp.int32),
            pltpu.VMEM((per_worker,), jnp.float32),
            pltpu.VMEM((SB,), jnp.int32),
            pltpu.VMEM((SB,), jnp.int32),
            pltpu.VMEM((SB, d), jnp.float32),
            pltpu.VMEM((SB, d), jnp.float32),
            pltpu.VMEM_SHARED((n, d), jnp.float32),
            pltpu.SemaphoreType.DMA,
            pltpu.SemaphoreType.DMA,
            pltpu.SemaphoreType.DMA,
            pltpu.SemaphoreType.DMA,
            pltpu.SemaphoreType.DMA,
            pltpu.SemaphoreType.DMA,
        ],
    )
    def sc_aggr(h_hbm, src_hbm, dst_hbm, w_hbm, out_hbm,
                src_all, w_all, dst_a, dst_b, rows_a, rows_b,
                acc_sp, gsem_a, gsem_b, ssem_a, ssem_b, dsem_a, dsem_b):
        zbuf = rows_a
        c = lax.axis_index("c")
        s = lax.axis_index("s")
        wid = c * N_SUB + s
        bufs = ((dst_a, rows_a, gsem_a, ssem_a, dsem_a),
                (dst_b, rows_b, gsem_b, ssem_b, dsem_b))
        ebase = wid * per_worker

        def z_body(i, _):
            for j in range(d // LANES):
                zbuf[i, pl.ds(j * LANES, LANES)] = jnp.zeros((LANES,), jnp.float32)
            return 0
        lax.fori_loop(0, zchunk, z_body, 0)
        row0 = s * rows_per_tile
        for k in range(n_zc):
            pltpu.sync_copy(zbuf.at[pl.ds(0, zchunk)],
                            acc_sp.at[pl.ds(row0 + k * zchunk, zchunk)])

        pltpu.sync_copy(src_hbm.at[pl.ds(ebase, per_worker)], src_all)
        pltpu.sync_copy(w_hbm.at[pl.ds(ebase, per_worker)], w_all)
        plsc.subcore_barrier()

        def start_dst(i, q):
            dst_v, _, _, _, dsem = bufs[q]
            pltpu.async_copy(dst_hbm.at[pl.ds(ebase + i * SB, SB)], dst_v, dsem)

        def wait_dst(q):
            dst_v, _, _, _, dsem = bufs[q]
            pltpu.make_async_copy(dst_hbm.at[pl.ds(0, SB)], dst_v, dsem).wait()

        def start_gather(i, q):
            _, rows_v, gsem, _, _ = bufs[q]
            pltpu.async_copy(h_hbm.at[src_all.at[pl.ds(i * SB, SB)]],
                             rows_v, gsem)

        def drain_gather(q):
            _, rows_v, gsem, _, _ = bufs[q]
            pltpu.make_async_copy(h_hbm.at[src_all.at[pl.ds(0, SB)]],
                                  rows_v, gsem).wait()

        def scale(i, q):
            _, rows_v, _, _, _ = bufs[q]

            def g_body(g, _):
                gb = g * LANES
                w16 = w_all[pl.ds(i * SB + gb, LANES)]
                for b in range(LANES):
                    wb = lax.gather(
                        w16, jnp.full((LANES, 1), b, jnp.int32),
                        lax.GatherDimensionNumbers(
                            offset_dims=(), collapsed_slice_dims=(0,),
                            start_index_map=(0,)),
                        slice_sizes=(1,),
                        mode=lax.GatherScatterMode.PROMISE_IN_BOUNDS)
                    for j in range(d // LANES):
                        sl = pl.ds(j * LANES, LANES)
                        rows_v[gb + b, sl] = rows_v[gb + b, sl] * wb
                return 0
            lax.fori_loop(0, SB // LANES, g_body, 0)

        def start_scatter(q):
            dst_v, rows_v, _, ssem, _ = bufs[q]
            pltpu.async_copy(rows_v, acc_sp.at[dst_v], ssem, add=True)

        def drain_scatter(q):
            dst_v, rows_v, _, ssem, _ = bufs[q]
            pltpu.make_async_copy(rows_v, acc_sp.at[dst_v], ssem).wait()

        start_dst(0, 0)
        start_gather(0, 0)

        def sb_body(i, _):
            p = lax.rem(i, 2)
            for q in range(2):
                @pl.when(p == q)
                def _():
                    @pl.when(i < n_sb - 1)
                    def _():
                        @pl.when(i >= 1)
                        def _():
                            drain_scatter(1 - q)
                        start_dst(i + 1, 1 - q)
                        start_gather(i + 1, 1 - q)
                    drain_gather(q)
                    scale(i, q)
                    wait_dst(q)
                    start_scatter(q)
            return 0
        lax.fori_loop(0, n_sb, sb_body, 0)
        drain_scatter((n_sb - 1) % 2)
        drain_scatter(n_sb % 2)

        plsc.subcore_barrier()
        for k in range(n_zc):
            r = row0 + k * zchunk
            pltpu.sync_copy(acc_sp.at[pl.ds(r, zchunk)], zbuf.at[pl.ds(0, zchunk)])
            pltpu.sync_copy(zbuf.at[pl.ds(0, zchunk)], out_hbm.at[c, pl.ds(r, zchunk)])

    return sc_aggr


@functools.lru_cache(maxsize=None)
def _build_tc_update(n, d):
    blk = 1024
    assert n % blk == 0

    def tc_body(a0_ref, a1_ref, h_ref, w1t_ref, w2t_ref, b_ref, out_ref):
        aggr = a0_ref[...] + a1_ref[...]
        z = jnp.dot(aggr, w1t_ref[...], preferred_element_type=jnp.float32)
        z = z + jnp.dot(h_ref[...], w2t_ref[...],
                        preferred_element_type=jnp.float32)
        out_ref[...] = jnp.tanh(z + b_ref[...])

    return pl.pallas_call(
        tc_body,
        grid=(n // blk,),
        in_specs=[
            pl.BlockSpec((blk, d), lambda i: (i, 0)),
            pl.BlockSpec((blk, d), lambda i: (i, 0)),
            pl.BlockSpec((blk, d), lambda i: (i, 0)),
            pl.BlockSpec((d, d), lambda i: (0, 0)),
            pl.BlockSpec((d, d), lambda i: (0, 0)),
            pl.BlockSpec((1, d), lambda i: (0, 0)),
        ],
        out_specs=pl.BlockSpec((blk, d), lambda i: (i, 0)),
        out_shape=jax.ShapeDtypeStruct((n, d), jnp.float32),
    )


def kernel(x, edge_index, edge_weights, Ws, bs):
    n, d = x.shape
    e = edge_weights.shape[0]
    num_layers = Ws.shape[0]

    n_pad = ((n + 2047) // 2048) * 2048
    chunk = N_WORKERS * SB
    e_pad = ((e + chunk - 1) // chunk) * chunk

    fill = (jnp.arange(e_pad - e, dtype=jnp.int32) % n_pad)
    src = jnp.concatenate([edge_index[0], fill])
    dst = jnp.concatenate([edge_index[1], fill])
    ew = jnp.concatenate(
        [edge_weights, jnp.zeros((e_pad - e,), jnp.float32)])

    w1t = jnp.transpose(Ws[:, :, :d], (0, 2, 1))
    w2t = jnp.transpose(Ws[:, :, d:], (0, 2, 1))
    b2 = bs.reshape(num_layers, 1, d)

    sc_aggr = _build_sc_aggregate(n_pad, d, e_pad)
    tc_update = _build_tc_update(n_pad, d)

    h = jnp.pad(x, ((0, n_pad - n), (0, 0)))
    for l in range(num_layers):
        parts = sc_aggr(h, src, dst, ew)
        h = tc_update(parts[0], parts[1], h, w1t[l], w2t[l], b2[l])
    return h[:n]


# speedup vs baseline: 11.1832x; 1.1255x over previous
import functools

import jax
import jax.numpy as jnp
from jax import lax
from jax.experimental import pallas as pl
from jax.experimental.pallas import tpu as pltpu
from jax.experimental.pallas import tpu_sc as plsc

N_CORES = 2
N_SUB = 16
N_WORKERS = N_CORES * N_SUB
LANES = 16
SB = 112


@functools.lru_cache(maxsize=None)
def _build_sc_aggregate(n, d, e_pad):
    per_worker = e_pad // N_WORKERS
    assert per_worker * N_WORKERS == e_pad
    n_sb = per_worker // SB
    assert n_sb * SB == per_worker and n_sb >= 3
    rows_per_tile = n // N_SUB
    zchunk = 80
    n_zc = rows_per_tile // zchunk
    assert n_zc * zchunk == rows_per_tile and zchunk <= SB

    mesh = plsc.VectorSubcoreMesh(core_axis_name="c", subcore_axis_name="s")

    NBUF = 3

    @functools.partial(
        pl.kernel,
        out_type=jax.ShapeDtypeStruct((N_CORES, n, d), jnp.float32),
        mesh=mesh,
        scratch_types=(
            [pltpu.VMEM((SB,), jnp.int32) for _ in range(NBUF)]
            + [pltpu.VMEM((SB,), jnp.float32) for _ in range(NBUF)]
            + [pltpu.VMEM((SB,), jnp.int32) for _ in range(NBUF)]
            + [pltpu.VMEM((SB, d), jnp.float32) for _ in range(NBUF)]
            + [pltpu.VMEM_SHARED((n, d), jnp.float32)]
            + [pltpu.SemaphoreType.DMA for _ in range(4 * NBUF)]
        ),
    )
    def sc_aggr(h_hbm, src_hbm, wgt_hbm, dst_hbm, out_hbm,
                sw0, sw1, sw2, wv0, wv1, wv2, dst0, dst1, dst2,
                rows0, rows1, rows2, acc_sp, *sems):
        zbuf = rows0
        c = lax.axis_index("c")
        s = lax.axis_index("s")
        wid = c * N_SUB + s
        sw = (sw0, sw1, sw2)
        wv = (wv0, wv1, wv2)
        dst = (dst0, dst1, dst2)
        rows = (rows0, rows1, rows2)
        wsem = sems[0:NBUF]
        dsem = sems[NBUF:2 * NBUF]
        gsem = sems[2 * NBUF:3 * NBUF]
        ssem = sems[3 * NBUF:4 * NBUF]
        sb_base = wid * n_sb
        ebase = wid * per_worker

        def z_body(i, _):
            for j in range(d // LANES):
                zbuf[i, pl.ds(j * LANES, LANES)] = jnp.zeros((LANES,), jnp.float32)
            return 0
        lax.fori_loop(0, zchunk, z_body, 0)
        row0 = s * rows_per_tile
        for k in range(n_zc):
            pltpu.sync_copy(zbuf.at[pl.ds(0, zchunk)],
                            acc_sp.at[pl.ds(row0 + k * zchunk, zchunk)])
        plsc.subcore_barrier()

        def start_idx(i, q):
            pltpu.async_copy(src_hbm.at[pl.ds(ebase + i * SB, SB)],
                             sw[q], wsem[q])
            pltpu.async_copy(wgt_hbm.at[pl.ds(ebase + i * SB, SB)],
                             wv[q], wsem[q])
            pltpu.async_copy(dst_hbm.at[pl.ds(ebase + i * SB, SB)],
                             dst[q], dsem[q])

        def wait_idx(q):
            pltpu.make_async_copy(src_hbm.at[pl.ds(0, SB)], sw[q],
                                  wsem[q]).wait()
            pltpu.make_async_copy(wgt_hbm.at[pl.ds(0, SB)], wv[q],
                                  wsem[q]).wait()

        def wait_dst(q):
            pltpu.make_async_copy(dst_hbm.at[pl.ds(0, SB)], dst[q],
                                  dsem[q]).wait()

        def start_gather(q):
            pltpu.async_copy(h_hbm.at[sw[q]], rows[q], gsem[q])

        def drain_gather(q):
            pltpu.make_async_copy(h_hbm.at[sw[q]], rows[q],
                                  gsem[q]).wait()

        def scale(q):
            rows_v = rows[q]

            def g_body(g, _):
                gb = g * LANES
                w16 = wv[q][pl.ds(gb, LANES)]
                for b in range(LANES):
                    wb = lax.gather(
                        w16, jnp.full((LANES, 1), b, jnp.int32),
                        lax.GatherDimensionNumbers(
                            offset_dims=(), collapsed_slice_dims=(0,),
                            start_index_map=(0,)),
                        slice_sizes=(1,),
                        mode=lax.GatherScatterMode.PROMISE_IN_BOUNDS)
                    for j in range(d // LANES):
                        sl = pl.ds(j * LANES, LANES)
                        rows_v[gb + b, sl] = rows_v[gb + b, sl] * wb
                return 0
            lax.fori_loop(0, SB // LANES, g_body, 0)

        def start_scatter(q):
            pltpu.async_copy(rows[q], acc_sp.at[dst[q]], ssem[q], add=True)

        def drain_scatter(q):
            pltpu.make_async_copy(rows[q], acc_sp.at[dst[q]], ssem[q]).wait()

        start_idx(0, 0)
        start_idx(1, 1)
        wait_idx(0)
        start_gather(0)
        wait_idx(1)
        start_gather(1)

        def sb_body(i, _):
            p = lax.rem(i, NBUF)
            for q in range(NBUF):
                @pl.when(p == q)
                def _():
                    r = (q + 2) % NBUF

                    @pl.when(i + 2 <= n_sb - 1)
                    def _():
                        start_idx(i + 2, r)
                    drain_gather(q)
                    scale(q)
                    wait_dst(q)
                    start_scatter(q)

                    @pl.when(i + 2 <= n_sb - 1)
                    def _():
                        @pl.when(i >= 1)
                        def _():
                            drain_scatter(r)
                        wait_idx(r)
                        start_gather(r)
            return 0
        lax.fori_loop(0, n_sb, sb_body, 0)
        for k in range(3):
            drain_scatter((n_sb - 3 + k) % NBUF)

        plsc.subcore_barrier()
        for k in range(n_zc):
            r = row0 + k * zchunk
            pltpu.sync_copy(acc_sp.at[pl.ds(r, zchunk)], zbuf.at[pl.ds(0, zchunk)])
            pltpu.sync_copy(zbuf.at[pl.ds(0, zchunk)], out_hbm.at[c, pl.ds(r, zchunk)])

    return sc_aggr


@functools.lru_cache(maxsize=None)
def _build_tc_update(n, d):
    blk = 1024
    assert n % blk == 0

    def tc_body(a0_ref, a1_ref, h_ref, w1t_ref, w2t_ref, b_ref, out_ref):
        aggr = a0_ref[...] + a1_ref[...]
        z = jnp.dot(aggr, w1t_ref[...], preferred_element_type=jnp.float32)
        z = z + jnp.dot(h_ref[...], w2t_ref[...],
                        preferred_element_type=jnp.float32)
        out_ref[...] = jnp.tanh(z + b_ref[...])

    return pl.pallas_call(
        tc_body,
        grid=(n // blk,),
        in_specs=[
            pl.BlockSpec((blk, d), lambda i: (i, 0)),
            pl.BlockSpec((blk, d), lambda i: (i, 0)),
            pl.BlockSpec((blk, d), lambda i: (i, 0)),
            pl.BlockSpec((d, d), lambda i: (0, 0)),
            pl.BlockSpec((d, d), lambda i: (0, 0)),
            pl.BlockSpec((1, d), lambda i: (0, 0)),
        ],
        out_specs=pl.BlockSpec((blk, d), lambda i: (i, 0)),
        out_shape=jax.ShapeDtypeStruct((n, d), jnp.float32),
    )


def kernel(x, edge_index, edge_weights, Ws, bs):
    n, d = x.shape
    e = edge_weights.shape[0]
    num_layers = Ws.shape[0]

    n_pad = ((n + 2047) // 2048) * 2048
    chunk = N_WORKERS * SB
    e_pad = ((e + chunk - 1) // chunk) * chunk

    fill = (jnp.arange(e_pad - e, dtype=jnp.int32) % n_pad)
    src = jnp.concatenate([edge_index[0], fill])
    dst = jnp.concatenate([edge_index[1], fill])
    ew = jnp.concatenate(
        [edge_weights, jnp.zeros((e_pad - e,), jnp.float32)])


    w1t = jnp.transpose(Ws[:, :, :d], (0, 2, 1))
    w2t = jnp.transpose(Ws[:, :, d:], (0, 2, 1))
    b2 = bs.reshape(num_layers, 1, d)

    sc_aggr = _build_sc_aggregate(n_pad, d, e_pad)
    tc_update = _build_tc_update(n_pad, d)

    h = jnp.pad(x, ((0, n_pad - n), (0, 0)))
    for l in range(num_layers):
        parts = sc_aggr(h, src, ew, dst)
        h = tc_update(parts[0], parts[1], h, w1t[l], w2t[l], b2[l])
    return h[:n]
